# baseline (speedup 1.0000x reference)
import jax
import jax.numpy as jnp
from jax.experimental import pallas as pl
from jax.experimental.pallas import tpu as pltpu
from jax.experimental.pallas import tpu_sc as plsc

_T = 2048
_D = 1024
_E = 8
_K = 2
_F = 2816
_FB = 1408
_NFC = _F // _FB
_R = 256
_NT = 24
_NS = _NT * _R
_SUB = 8
_DC = _D // _SUB


def _cumsum_rows_excl(a):
    acc = a
    sh = 1
    while sh < a.shape[0]:
        shifted = jnp.concatenate(
            [jnp.zeros((sh, a.shape[1]), a.dtype), acc[:-sh]], axis=0)
        acc = acc + shifted
        sh *= 2
    return acc - a


def _router_dispatch_kernel(x_ref, wr_ref, logits_ref, rw_ref, idx_ref, w_ref,
                            dest_ref, d1x_ref, d2x_ref, tile_e_ref,
                            w1w_ref, w2w_ref):
    x = x_ref[...]
    wr = wr_ref[...]
    logits = jax.lax.dot_general(
        x, wr, (((1,), (0,)), ((), ())),
        preferred_element_type=jnp.float32,
    )
    rw = jax.nn.softmax(logits, axis=-1)
    eidx = jax.lax.broadcasted_iota(jnp.int32, (_T, _E), 1)
    m1 = jnp.max(rw, axis=1, keepdims=True)
    i1 = jnp.min(jnp.where(rw >= m1, eidx, _E), axis=1, keepdims=True)
    masked = jnp.where(eidx == i1, -jnp.inf, rw)
    m2 = jnp.max(masked, axis=1, keepdims=True)
    i2 = jnp.min(jnp.where(masked >= m2, eidx, _E), axis=1, keepdims=True)
    wsum = m1 + m2
    w1 = m1 / wsum
    w2 = m2 / wsum
    logits_ref[...] = logits
    rw_ref[...] = rw
    idx_ref[...] = jnp.concatenate([i1, i2], axis=1)
    w_ref[...] = jnp.concatenate([w1, w2], axis=1)
    w1w_ref[...] = jnp.broadcast_to(w1, (_T, 16 * _SUB))
    w2w_ref[...] = jnp.broadcast_to(w2, (_T, 16 * _SUB))

    oh1 = (eidx == i1).astype(jnp.float32)
    oh2 = (eidx == i2).astype(jnp.float32)
    r1 = _cumsum_rows_excl(oh1)
    r2 = _cumsum_rows_excl(oh2)
    t1 = jnp.sum(oh1, axis=0, keepdims=True)
    t2 = jnp.sum(oh2, axis=0, keepdims=True)
    cnt = t1 + t2
    pad = jnp.ceil(cnt * (1.0 / _R)) * _R
    incl = pad
    sh = 1
    while sh < _E:
        incl = incl + jnp.concatenate(
            [jnp.zeros((1, sh), jnp.float32), incl[:, :-sh]], axis=1)
        sh *= 2
    off = incl - pad
    d1 = jnp.sum(oh1 * (off + r1), axis=1, keepdims=True)
    d2 = jnp.sum(oh2 * (off + t1 + r2), axis=1, keepdims=True)
    dest_ref[...] = jnp.concatenate([d1, d2], axis=1).astype(jnp.int32)
    sub = jax.lax.broadcasted_iota(jnp.int32, (_T, _SUB), 1)
    d1x_ref[...] = d1.astype(jnp.int32) * _SUB + sub
    d2x_ref[...] = d2.astype(jnp.int32) * _SUB + sub
    tstart = jax.lax.broadcasted_iota(
        jnp.int32, (_NT, _E), 0).astype(jnp.float32) * float(_R)
    te = jnp.sum((tstart >= incl).astype(jnp.float32), axis=1, keepdims=True)
    tile_e_ref[...] = jnp.minimum(te, float(_E - 1)).astype(jnp.int32)


def _router_dispatch(x, wr):
    return pl.pallas_call(
        _router_dispatch_kernel,
        out_shape=(
            jax.ShapeDtypeStruct((_T, _E), jnp.float32),
            jax.ShapeDtypeStruct((_T, _E), jnp.float32),
            jax.ShapeDtypeStruct((_T, _K), jnp.int32),
            jax.ShapeDtypeStruct((_T, _K), jnp.float32),
            jax.ShapeDtypeStruct((_T, _K), jnp.int32),
            jax.ShapeDtypeStruct((_T, _SUB), jnp.int32),
            jax.ShapeDtypeStruct((_T, _SUB), jnp.int32),
            jax.ShapeDtypeStruct((_NT, 1), jnp.int32),
            jax.ShapeDtypeStruct((_T, 16 * _SUB), jnp.float32),
            jax.ShapeDtypeStruct((_T, 16 * _SUB), jnp.float32),
        ),
    )(x, wr)


def _sc_mesh():
    return plsc.VectorSubcoreMesh(
        core_axis_name="core", subcore_axis_name="subcore")


_SW = 128


def _sc_scatter(x8, d1x, d2x):
    @pl.kernel(out_type=jax.ShapeDtypeStruct((_SUB * _NS, _DC), jnp.float32),
               mesh=_sc_mesh())
    def k(x_hbm, d1_hbm, d2_hbm, xs_hbm):
        def body(x_vmem, d1_vmem, d2_vmem):
            pltpu.sync_copy(x_vmem, xs_hbm.at[d1_vmem.at[0]])
            pltpu.sync_copy(x_vmem, xs_hbm.at[d2_vmem.at[0]])

        pltpu.emit_pipeline(
            body,
            grid=(_SUB * _T // _SW,),
            in_specs=[
                pl.BlockSpec((_SW, _DC), lambda i: (i, 0)),
                pl.BlockSpec((1, _SW), lambda i: (0, i)),
                pl.BlockSpec((1, _SW), lambda i: (0, i)),
            ],
            out_specs=[],
            core_axis_name=("core", "subcore"),
            dimension_semantics=(pltpu.PARALLEL,),
        )(x_hbm, d1_hbm, d2_hbm)

    return k(x8, d1x, d2x)


def _sc_combine(y8, d1x, d2x, w1x, w2x):
    @pl.kernel(out_type=jax.ShapeDtypeStruct((_SUB * _T, _DC), jnp.float32),
               mesh=_sc_mesh(),
               scratch_types=[pltpu.VMEM((_SW, _DC), jnp.float32)])
    def k(y_hbm, d1_hbm, d2_hbm, w1_hbm, w2_hbm, o_hbm, a_ref):
        def body(d1_vmem, d2_vmem, w1_vmem, w2_vmem, o_vmem):
            pltpu.sync_copy(y_hbm.at[d1_vmem.at[0]], a_ref)

            @pl.loop(0, _SW)
            def _(t):
                @pl.loop(0, _DC, step=16)
                def _(c):
                    sl = (pl.ds(t, 1), pl.ds(c, 16))
                    wsl = (pl.ds(t, 1), pl.ds(0, 16))
                    o_vmem.at[*sl][...] = (
                        w1_vmem.at[*wsl][...] * a_ref.at[*sl][...])

            pltpu.sync_copy(y_hbm.at[d2_vmem.at[0]], a_ref)

            @pl.loop(0, _SW)
            def _(t):
                @pl.loop(0, _DC, step=16)
                def _(c):
                    sl = (pl.ds(t, 1), pl.ds(c, 16))
                    wsl = (pl.ds(t, 1), pl.ds(0, 16))
                    o_vmem.at[*sl][...] = (
                        o_vmem.at[*sl][...]
                        + w2_vmem.at[*wsl][...] * a_ref.at[*sl][...])

        pltpu.emit_pipeline(
            body,
            grid=(_SUB * _T // _SW,),
            in_specs=[
                pl.BlockSpec((1, _SW), lambda i: (0, i)),
                pl.BlockSpec((1, _SW), lambda i: (0, i)),
                pl.BlockSpec((_SW, 16), lambda i: (i, 0)),
                pl.BlockSpec((_SW, 16), lambda i: (i, 0)),
            ],
            out_specs=[pl.BlockSpec((_SW, _DC), lambda i: (i, 0))],
            core_axis_name=("core", "subcore"),
            dimension_semantics=(pltpu.PARALLEL,),
        )(d1_hbm, d2_hbm, w1_hbm, w2_hbm, o_hbm)

    return k(y8, d1x, d2x, w1x, w2x)


def _e1_kernel(te_ref, xs_ref, wg_ref, wu_ref, h_ref):
    xs = xs_ref[...].astype(jnp.bfloat16)
    g = jnp.dot(xs, wg_ref[0].astype(jnp.bfloat16),
                preferred_element_type=jnp.float32)
    u = jnp.dot(xs, wu_ref[0].astype(jnp.bfloat16),
                preferred_element_type=jnp.float32)
    h_ref[...] = (jax.nn.silu(g) * u).astype(jnp.bfloat16)


def _e1(tile_e, xs, wg, wu):
    grid_spec = pltpu.PrefetchScalarGridSpec(
        num_scalar_prefetch=1,
        grid=(_NFC, _NT),
        in_specs=[
            pl.BlockSpec((_R, _D), lambda f, i, te: (i, 0)),
            pl.BlockSpec((1, _D, _FB), lambda f, i, te: (te[i], 0, f)),
            pl.BlockSpec((1, _D, _FB), lambda f, i, te: (te[i], 0, f)),
        ],
        out_specs=pl.BlockSpec((_R, _FB), lambda f, i, te: (i, f)),
    )
    return pl.pallas_call(
        _e1_kernel,
        grid_spec=grid_spec,
        out_shape=jax.ShapeDtypeStruct((_NS, _F), jnp.bfloat16),
        compiler_params=pltpu.CompilerParams(
            dimension_semantics=("arbitrary", "arbitrary")),
    )(tile_e, xs, wg, wu)


def _e2_kernel(te_ref, h_ref, wd_ref, y_ref):
    fc = pl.program_id(0)
    i = pl.program_id(1)
    y = jnp.dot(h_ref[...], wd_ref[0].astype(jnp.bfloat16),
                preferred_element_type=jnp.float32)
    row = i * _R

    @pl.when(fc == 0)
    def _():
        y_ref[pl.ds(row, _R), :] = y

    @pl.when(fc != 0)
    def _():
        y_ref[pl.ds(row, _R), :] += y


def _e2(tile_e, h, wd):
    grid_spec = pltpu.PrefetchScalarGridSpec(
        num_scalar_prefetch=1,
        grid=(_NFC, _NT),
        in_specs=[
            pl.BlockSpec((_R, _FB), lambda f, i, te: (i, f)),
            pl.BlockSpec((1, _FB, _D), lambda f, i, te: (te[i], f, 0)),
        ],
        out_specs=pl.BlockSpec((_NS, _D), lambda f, i, te: (0, 0)),
    )
    return pl.pallas_call(
        _e2_kernel,
        grid_spec=grid_spec,
        out_shape=jax.ShapeDtypeStruct((_NS, _D), jnp.float32),
        compiler_params=pltpu.CompilerParams(
            dimension_semantics=("arbitrary", "arbitrary")),
    )(tile_e, h, wd)


@jax.jit
def kernel(hidden_states, Wr, Wg, Wu, Wd):
    b, s, d = hidden_states.shape
    x = hidden_states.reshape(s, d)
    (logits, rw, idx, w, dest, d1x, d2x, tile_e,
     w1w, w2w) = _router_dispatch(x, Wr)
    d1r = d1x.reshape(1, _SUB * _T)
    d2r = d2x.reshape(1, _SUB * _T)
    w1r = w1w.reshape(_SUB * _T, 16)
    w2r = w2w.reshape(_SUB * _T, 16)
    tile_e1 = tile_e.reshape(_NT)
    xs8 = _sc_scatter(x.reshape(_SUB * _T, _DC), d1r, d2r)
    h = _e1(tile_e1, xs8.reshape(_NS, _D), Wg, Wu)
    y = _e2(tile_e1, h, Wd)
    out8 = _sc_combine(y.reshape(_SUB * _NS, _DC), d1r, d2r, w1r, w2r)
    out = out8.reshape(_T, _D)
    return (out.reshape(b, s, d),
            logits.reshape(b, s, _E),
            idx.reshape(b, s, _K),
            w.reshape(b, s, _K),
            rw.reshape(b, s, _E))


# speedup vs baseline: 1.1236x; 1.1236x over previous
import jax
import jax.numpy as jnp
from jax.experimental import pallas as pl
from jax.experimental.pallas import tpu as pltpu
from jax.experimental.pallas import tpu_sc as plsc

_T = 2048
_D = 1024
_E = 8
_K = 2
_F = 2816
_FB = 1408
_NFC = _F // _FB
_R = 256
_NT = 24
_NS = _NT * _R
_SUB = 8
_DC = _D // _SUB
_SW = 128


def _cumsum_rows_excl(a):
    acc = a
    sh = 1
    while sh < a.shape[0]:
        shifted = jnp.concatenate(
            [jnp.zeros((sh, a.shape[1]), a.dtype), acc[:-sh]], axis=0)
        acc = acc + shifted
        sh *= 2
    return acc - a


def _router_dispatch_kernel(x_ref, wr_ref, logits_ref, rw_ref, idx_ref, w_ref,
                            dest_ref, d1x_ref, d2x_ref, tile_e_ref,
                            wcat_ref):
    x = x_ref[...]
    wr = wr_ref[...]
    logits = jax.lax.dot_general(
        x, wr, (((1,), (0,)), ((), ())),
        preferred_element_type=jnp.float32,
    )
    rw = jax.nn.softmax(logits, axis=-1)
    eidx = jax.lax.broadcasted_iota(jnp.int32, (_T, _E), 1)
    m1 = jnp.max(rw, axis=1, keepdims=True)
    i1 = jnp.min(jnp.where(rw >= m1, eidx, _E), axis=1, keepdims=True)
    masked = jnp.where(eidx == i1, -jnp.inf, rw)
    m2 = jnp.max(masked, axis=1, keepdims=True)
    i2 = jnp.min(jnp.where(masked >= m2, eidx, _E), axis=1, keepdims=True)
    wsum = m1 + m2
    w1 = m1 / wsum
    w2 = m2 / wsum
    logits_ref[...] = logits
    rw_ref[...] = rw
    idx_ref[...] = jnp.concatenate([i1, i2], axis=1)
    w_ref[...] = jnp.concatenate([w1, w2], axis=1)
    wcat_ref[...] = jnp.concatenate(
        [jnp.broadcast_to(w1, (_T, 128)), jnp.broadcast_to(w2, (_T, 128))],
        axis=0)

    oh1 = (eidx == i1).astype(jnp.float32)
    oh2 = (eidx == i2).astype(jnp.float32)
    r1 = _cumsum_rows_excl(oh1)
    r2 = _cumsum_rows_excl(oh2)
    t1 = jnp.sum(oh1, axis=0, keepdims=True)
    t2 = jnp.sum(oh2, axis=0, keepdims=True)
    cnt = t1 + t2
    pad = jnp.ceil(cnt * (1.0 / _R)) * _R
    incl = pad
    sh = 1
    while sh < _E:
        incl = incl + jnp.concatenate(
            [jnp.zeros((1, sh), jnp.float32), incl[:, :-sh]], axis=1)
        sh *= 2
    off = incl - pad
    d1 = jnp.sum(oh1 * (off + r1), axis=1, keepdims=True)
    d2 = jnp.sum(oh2 * (off + t1 + r2), axis=1, keepdims=True)
    dest_ref[...] = jnp.concatenate([d1, d2], axis=1).astype(jnp.int32)
    sub = jax.lax.broadcasted_iota(jnp.int32, (_T, _SUB), 1)
    d1x_ref[...] = d1.astype(jnp.int32) * _SUB + sub
    d2x_ref[...] = d2.astype(jnp.int32) * _SUB + sub
    tstart = jax.lax.broadcasted_iota(
        jnp.int32, (_NT, _E), 0).astype(jnp.float32) * float(_R)
    te = jnp.sum((tstart >= incl).astype(jnp.float32), axis=1, keepdims=True)
    tile_e_ref[...] = jnp.minimum(te, float(_E - 1)).astype(jnp.int32)


def _router_dispatch(x, wr):
    return pl.pallas_call(
        _router_dispatch_kernel,
        out_shape=(
            jax.ShapeDtypeStruct((_T, _E), jnp.float32),
            jax.ShapeDtypeStruct((_T, _E), jnp.float32),
            jax.ShapeDtypeStruct((_T, _K), jnp.int32),
            jax.ShapeDtypeStruct((_T, _K), jnp.float32),
            jax.ShapeDtypeStruct((_T, _K), jnp.int32),
            jax.ShapeDtypeStruct((_T, _SUB), jnp.int32),
            jax.ShapeDtypeStruct((_T, _SUB), jnp.int32),
            jax.ShapeDtypeStruct((_NT, 1), jnp.int32),
            jax.ShapeDtypeStruct((2 * _T, 128), jnp.float32),
        ),
    )(x, wr)


def _sc_mesh():
    return plsc.VectorSubcoreMesh(
        core_axis_name="core", subcore_axis_name="subcore")


def _sc_scatter(x8, d1x, d2x, wcat, dT):
    @pl.kernel(out_type=(jax.ShapeDtypeStruct((_SUB * _NS, _DC), jnp.float32),
                         jax.ShapeDtypeStruct((_NS, 128), jnp.float32)),
               mesh=_sc_mesh())
    def k(x_hbm, d1_hbm, d2_hbm, w_hbm, dt_hbm, xs_hbm, ws_hbm):
        def body(x_vmem, d1_vmem, d2_vmem):
            pltpu.sync_copy(x_vmem, xs_hbm.at[d1_vmem.at[0]])
            pltpu.sync_copy(x_vmem, xs_hbm.at[d2_vmem.at[0]])

        pltpu.emit_pipeline(
            body,
            grid=(_SUB * _T // _SW,),
            in_specs=[
                pl.BlockSpec((_SW, _DC), lambda i: (i, 0)),
                pl.BlockSpec((1, _SW), lambda i: (0, i)),
                pl.BlockSpec((1, _SW), lambda i: (0, i)),
            ],
            out_specs=[],
            core_axis_name=("core", "subcore"),
            dimension_semantics=(pltpu.PARALLEL,),
        )(x_hbm, d1_hbm, d2_hbm)

        def wbody(w_vmem, dt_vmem):
            pltpu.sync_copy(w_vmem, ws_hbm.at[dt_vmem.at[0]])

        nblk = _T // _SW
        pltpu.emit_pipeline(
            wbody,
            grid=(2 * nblk,),
            in_specs=[
                pl.BlockSpec((_SW, 128), lambda i: (i, 0)),
                pl.BlockSpec((1, _SW), lambda i: (i // nblk, i % nblk)),
            ],
            out_specs=[],
            core_axis_name=("core", "subcore"),
            dimension_semantics=(pltpu.PARALLEL,),
        )(w_hbm, dt_hbm)

    return k(x8, d1x, d2x, wcat, dT)


def _sc_combine(y8, d1x, d2x):
    @pl.kernel(out_type=jax.ShapeDtypeStruct((_SUB * _T, _DC), jnp.float32),
               mesh=_sc_mesh(),
               scratch_types=[pltpu.VMEM((_SW, _DC), jnp.float32),
                              pltpu.VMEM((_SW, _DC), jnp.float32)])
    def k(y_hbm, d1_hbm, d2_hbm, o_hbm, a_ref, b_ref):
        def body(d1_vmem, d2_vmem, o_vmem):
            pltpu.sync_copy(y_hbm.at[d1_vmem.at[0]], a_ref)
            pltpu.sync_copy(y_hbm.at[d2_vmem.at[0]], b_ref)

            @pl.loop(0, _SW, step=8)
            def _(t):
                @pl.loop(0, _DC, step=16)
                def _(c):
                    sl = (pl.ds(t, 8), pl.ds(c, 16))
                    o_vmem.at[*sl][...] = (
                        a_ref.at[*sl][...] + b_ref.at[*sl][...])

        pltpu.emit_pipeline(
            body,
            grid=(_SUB * _T // _SW,),
            in_specs=[
                pl.BlockSpec((1, _SW), lambda i: (0, i)),
                pl.BlockSpec((1, _SW), lambda i: (0, i)),
            ],
            out_specs=[pl.BlockSpec((_SW, _DC), lambda i: (i, 0))],
            core_axis_name=("core", "subcore"),
            dimension_semantics=(pltpu.PARALLEL,),
        )(d1_hbm, d2_hbm, o_hbm)

    return k(y8, d1x, d2x)


def _e1_kernel(te_ref, xs_ref, wg_ref, wu_ref, ws_ref, h_ref):
    xs = xs_ref[...].astype(jnp.bfloat16)
    g = jnp.dot(xs, wg_ref[0].astype(jnp.bfloat16),
                preferred_element_type=jnp.float32)
    u = jnp.dot(xs, wu_ref[0].astype(jnp.bfloat16),
                preferred_element_type=jnp.float32)
    h_ref[...] = (jax.nn.silu(g) * u * ws_ref[:, 0:1]).astype(jnp.bfloat16)


def _e1(tile_e, xs, wg, wu, ws):
    grid_spec = pltpu.PrefetchScalarGridSpec(
        num_scalar_prefetch=1,
        grid=(_NFC, _NT),
        in_specs=[
            pl.BlockSpec((_R, _D), lambda f, i, te: (i, 0)),
            pl.BlockSpec((1, _D, _FB), lambda f, i, te: (te[i], 0, f)),
            pl.BlockSpec((1, _D, _FB), lambda f, i, te: (te[i], 0, f)),
            pl.BlockSpec((_R, 128), lambda f, i, te: (i, 0)),
        ],
        out_specs=pl.BlockSpec((_R, _FB), lambda f, i, te: (i, f)),
    )
    return pl.pallas_call(
        _e1_kernel,
        grid_spec=grid_spec,
        out_shape=jax.ShapeDtypeStruct((_NS, _F), jnp.bfloat16),
        compiler_params=pltpu.CompilerParams(
            dimension_semantics=("arbitrary", "arbitrary")),
    )(tile_e, xs, wg, wu, ws)


def _e2_kernel(te_ref, h_ref, wd_ref, y_ref):
    fc = pl.program_id(0)
    i = pl.program_id(1)
    y = jnp.dot(h_ref[...], wd_ref[0].astype(jnp.bfloat16),
                preferred_element_type=jnp.float32)
    row = i * _R

    @pl.when(fc == 0)
    def _():
        y_ref[pl.ds(row, _R), :] = y

    @pl.when(fc != 0)
    def _():
        y_ref[pl.ds(row, _R), :] += y


def _e2(tile_e, h, wd):
    grid_spec = pltpu.PrefetchScalarGridSpec(
        num_scalar_prefetch=1,
        grid=(_NFC, _NT),
        in_specs=[
            pl.BlockSpec((_R, _FB), lambda f, i, te: (i, f)),
            pl.BlockSpec((1, _FB, _D), lambda f, i, te: (te[i], f, 0)),
        ],
        out_specs=pl.BlockSpec((_NS, _D), lambda f, i, te: (0, 0)),
    )
    return pl.pallas_call(
        _e2_kernel,
        grid_spec=grid_spec,
        out_shape=jax.ShapeDtypeStruct((_NS, _D), jnp.float32),
        compiler_params=pltpu.CompilerParams(
            dimension_semantics=("arbitrary", "arbitrary")),
    )(tile_e, h, wd)


@jax.jit
def kernel(hidden_states, Wr, Wg, Wu, Wd):
    b, s, d = hidden_states.shape
    x = hidden_states.reshape(s, d)
    (logits, rw, idx, w, dest, d1x, d2x, tile_e,
     wcat) = _router_dispatch(x, Wr)
    d1r = d1x.reshape(1, _SUB * _T)
    d2r = d2x.reshape(1, _SUB * _T)
    dT = dest.T
    tile_e1 = tile_e.reshape(_NT)
    xs8, ws = _sc_scatter(x.reshape(_SUB * _T, _DC), d1r, d2r, wcat, dT)
    h = _e1(tile_e1, xs8.reshape(_NS, _D), Wg, Wu, ws)
    y = _e2(tile_e1, h, Wd)
    out8 = _sc_combine(y.reshape(_SUB * _NS, _DC), d1r, d2r)
    out = out8.reshape(_T, _D)
    return (out.reshape(b, s, d),
            logits.reshape(b, s, _E),
            idx.reshape(b, s, _K),
            w.reshape(b, s, _K),
            rw.reshape(b, s, _E))


# speedup vs baseline: 1.1543x; 1.0273x over previous
import jax
import jax.numpy as jnp
from jax.experimental import pallas as pl
from jax.experimental.pallas import tpu as pltpu
from jax.experimental.pallas import tpu_sc as plsc

_T = 2048
_D = 1024
_E = 8
_K = 2
_F = 2816
_FB = 1408
_NFC = _F // _FB
_R = 256
_NT = 24
_NS = _NT * _R
_SUB = 8
_DC = _D // _SUB
_SW = 128


def _cumsum_rows_excl(a):
    acc = a
    sh = 1
    while sh < a.shape[0]:
        shifted = jnp.concatenate(
            [jnp.zeros((sh, a.shape[1]), a.dtype), acc[:-sh]], axis=0)
        acc = acc + shifted
        sh *= 2
    return acc - a


def _router_dispatch_kernel(x_ref, wr_ref, logits_ref, rw_ref, idx_ref, w_ref,
                            dest_ref, d1x_ref, d2x_ref, tile_e_ref,
                            wcat_ref):
    x = x_ref[...]
    wr = wr_ref[...]
    logits = jax.lax.dot_general(
        x, wr, (((1,), (0,)), ((), ())),
        preferred_element_type=jnp.float32,
    )
    rw = jax.nn.softmax(logits, axis=-1)
    eidx = jax.lax.broadcasted_iota(jnp.int32, (_T, _E), 1)
    m1 = jnp.max(rw, axis=1, keepdims=True)
    i1 = jnp.min(jnp.where(rw >= m1, eidx, _E), axis=1, keepdims=True)
    masked = jnp.where(eidx == i1, -jnp.inf, rw)
    m2 = jnp.max(masked, axis=1, keepdims=True)
    i2 = jnp.min(jnp.where(masked >= m2, eidx, _E), axis=1, keepdims=True)
    wsum = m1 + m2
    w1 = m1 / wsum
    w2 = m2 / wsum
    logits_ref[...] = logits
    rw_ref[...] = rw
    idx_ref[...] = jnp.concatenate([i1, i2], axis=1)
    w_ref[...] = jnp.concatenate([w1, w2], axis=1)
    wcat_ref[...] = jnp.concatenate(
        [jnp.broadcast_to(w1, (_T, 128)), jnp.broadcast_to(w2, (_T, 128))],
        axis=0)

    oh1 = (eidx == i1).astype(jnp.float32)
    oh2 = (eidx == i2).astype(jnp.float32)
    r1 = _cumsum_rows_excl(oh1)
    r2 = _cumsum_rows_excl(oh2)
    t1 = jnp.sum(oh1, axis=0, keepdims=True)
    t2 = jnp.sum(oh2, axis=0, keepdims=True)
    cnt = t1 + t2
    pad = jnp.ceil(cnt * (1.0 / _R)) * _R
    incl = pad
    sh = 1
    while sh < _E:
        incl = incl + jnp.concatenate(
            [jnp.zeros((1, sh), jnp.float32), incl[:, :-sh]], axis=1)
        sh *= 2
    off = incl - pad
    d1 = jnp.sum(oh1 * (off + r1), axis=1, keepdims=True)
    d2 = jnp.sum(oh2 * (off + t1 + r2), axis=1, keepdims=True)
    dest_ref[...] = jnp.concatenate([d1, d2], axis=1).astype(jnp.int32)
    sub = jax.lax.broadcasted_iota(jnp.int32, (_T, _SUB), 1)
    d1x_ref[...] = d1.astype(jnp.int32) * _SUB + sub
    d2x_ref[...] = d2.astype(jnp.int32) * _SUB + sub
    tstart = jax.lax.broadcasted_iota(
        jnp.int32, (_NT + 1, _E), 0).astype(jnp.float32) * float(_R)
    te = jnp.sum((tstart >= incl).astype(jnp.float32), axis=1, keepdims=True)
    te = jnp.minimum(te, float(_E - 1))
    ntu = incl[0:1, _E - 1:_E] * (1.0 / _R)
    row_i = jax.lax.broadcasted_iota(jnp.int32, (_NT + 1, 1), 0)
    tile_e_ref[...] = jnp.where(row_i < _NT, te, ntu).astype(jnp.int32)


def _router_dispatch(x, wr):
    return pl.pallas_call(
        _router_dispatch_kernel,
        out_shape=(
            jax.ShapeDtypeStruct((_T, _E), jnp.float32),
            jax.ShapeDtypeStruct((_T, _E), jnp.float32),
            jax.ShapeDtypeStruct((_T, _K), jnp.int32),
            jax.ShapeDtypeStruct((_T, _K), jnp.float32),
            jax.ShapeDtypeStruct((_T, _K), jnp.int32),
            jax.ShapeDtypeStruct((_T, _SUB), jnp.int32),
            jax.ShapeDtypeStruct((_T, _SUB), jnp.int32),
            jax.ShapeDtypeStruct((_NT + 1, 1), jnp.int32),
            jax.ShapeDtypeStruct((2 * _T, 128), jnp.float32),
        ),
    )(x, wr)


def _sc_mesh():
    return plsc.VectorSubcoreMesh(
        core_axis_name="core", subcore_axis_name="subcore")


def _sc_scatter(x8, d1x, d2x, wcat, dT):
    @pl.kernel(out_type=(jax.ShapeDtypeStruct((_SUB * _NS, _DC), jnp.float32),
                         jax.ShapeDtypeStruct((_NS, 128), jnp.float32)),
               mesh=_sc_mesh())
    def k(x_hbm, d1_hbm, d2_hbm, w_hbm, dt_hbm, xs_hbm, ws_hbm):
        def body(x_vmem, d1_vmem, d2_vmem):
            pltpu.sync_copy(x_vmem, xs_hbm.at[d1_vmem.at[0]])
            pltpu.sync_copy(x_vmem, xs_hbm.at[d2_vmem.at[0]])

        pltpu.emit_pipeline(
            body,
            grid=(_SUB * _T // _SW,),
            in_specs=[
                pl.BlockSpec((_SW, _DC), lambda i: (i, 0)),
                pl.BlockSpec((1, _SW), lambda i: (0, i)),
                pl.BlockSpec((1, _SW), lambda i: (0, i)),
            ],
            out_specs=[],
            core_axis_name=("core", "subcore"),
            dimension_semantics=(pltpu.PARALLEL,),
        )(x_hbm, d1_hbm, d2_hbm)

        def wbody(w_vmem, dt_vmem):
            pltpu.sync_copy(w_vmem, ws_hbm.at[dt_vmem.at[0]])

        nblk = _T // _SW
        pltpu.emit_pipeline(
            wbody,
            grid=(2 * nblk,),
            in_specs=[
                pl.BlockSpec((_SW, 128), lambda i: (i, 0)),
                pl.BlockSpec((1, _SW), lambda i: (i // nblk, i % nblk)),
            ],
            out_specs=[],
            core_axis_name=("core", "subcore"),
            dimension_semantics=(pltpu.PARALLEL,),
        )(w_hbm, dt_hbm)

    return k(x8, d1x, d2x, wcat, dT)


def _sc_combine(y8, d1x, d2x):
    @pl.kernel(out_type=jax.ShapeDtypeStruct((_SUB * _T, _DC), jnp.float32),
               mesh=_sc_mesh(),
               scratch_types=[pltpu.VMEM((_SW, _DC), jnp.float32),
                              pltpu.VMEM((_SW, _DC), jnp.float32)])
    def k(y_hbm, d1_hbm, d2_hbm, o_hbm, a_ref, b_ref):
        def body(d1_vmem, d2_vmem, o_vmem):
            pltpu.sync_copy(y_hbm.at[d1_vmem.at[0]], a_ref)
            pltpu.sync_copy(y_hbm.at[d2_vmem.at[0]], b_ref)

            @pl.loop(0, _SW, step=8)
            def _(t):
                @pl.loop(0, _DC, step=16)
                def _(c):
                    sl = (pl.ds(t, 8), pl.ds(c, 16))
                    o_vmem.at[*sl][...] = (
                        a_ref.at[*sl][...] + b_ref.at[*sl][...])

        pltpu.emit_pipeline(
            body,
            grid=(_SUB * _T // _SW,),
            in_specs=[
                pl.BlockSpec((1, _SW), lambda i: (0, i)),
                pl.BlockSpec((1, _SW), lambda i: (0, i)),
            ],
            out_specs=[pl.BlockSpec((_SW, _DC), lambda i: (i, 0))],
            core_axis_name=("core", "subcore"),
            dimension_semantics=(pltpu.PARALLEL,),
        )(d1_hbm, d2_hbm, o_hbm)

    return k(y8, d1x, d2x)


def _e1_kernel(te_ref, xs_ref, wg_ref, wu_ref, ws_ref, h_ref):
    @pl.when(pl.program_id(1) < te_ref[_NT])
    def _():
        xs = xs_ref[...].astype(jnp.bfloat16)
        g = jnp.dot(xs, wg_ref[0].astype(jnp.bfloat16),
                    preferred_element_type=jnp.float32)
        u = jnp.dot(xs, wu_ref[0].astype(jnp.bfloat16),
                    preferred_element_type=jnp.float32)
        h_ref[...] = (jax.nn.silu(g) * u * ws_ref[:, 0:1]).astype(jnp.bfloat16)


def _e1(tile_e, xs, wg, wu, ws):
    grid_spec = pltpu.PrefetchScalarGridSpec(
        num_scalar_prefetch=1,
        grid=(_NFC, _NT),
        in_specs=[
            pl.BlockSpec((_R, _D), lambda f, i, te: (i, 0)),
            pl.BlockSpec((1, _D, _FB), lambda f, i, te: (te[i], 0, f)),
            pl.BlockSpec((1, _D, _FB), lambda f, i, te: (te[i], 0, f)),
            pl.BlockSpec((_R, 128), lambda f, i, te: (i, 0)),
        ],
        out_specs=pl.BlockSpec((_R, _FB), lambda f, i, te: (i, f)),
    )
    return pl.pallas_call(
        _e1_kernel,
        grid_spec=grid_spec,
        out_shape=jax.ShapeDtypeStruct((_NS, _F), jnp.bfloat16),
        compiler_params=pltpu.CompilerParams(
            dimension_semantics=("arbitrary", "arbitrary")),
    )(tile_e, xs, wg, wu, ws)


def _e2_kernel(te_ref, h_ref, wd_ref, y_ref):
    fc = pl.program_id(0)
    i = pl.program_id(1)

    @pl.when(i < te_ref[_NT])
    def _():
        y = jnp.dot(h_ref[...], wd_ref[0].astype(jnp.bfloat16),
                    preferred_element_type=jnp.float32)
        row = i * _R

        @pl.when(fc == 0)
        def _():
            y_ref[pl.ds(row, _R), :] = y

        @pl.when(fc != 0)
        def _():
            y_ref[pl.ds(row, _R), :] += y


def _e2(tile_e, h, wd):
    grid_spec = pltpu.PrefetchScalarGridSpec(
        num_scalar_prefetch=1,
        grid=(_NFC, _NT),
        in_specs=[
            pl.BlockSpec((_R, _FB), lambda f, i, te: (i, f)),
            pl.BlockSpec((1, _FB, _D), lambda f, i, te: (te[i], f, 0)),
        ],
        out_specs=pl.BlockSpec((_NS, _D), lambda f, i, te: (0, 0)),
    )
    return pl.pallas_call(
        _e2_kernel,
        grid_spec=grid_spec,
        out_shape=jax.ShapeDtypeStruct((_NS, _D), jnp.float32),
        compiler_params=pltpu.CompilerParams(
            dimension_semantics=("arbitrary", "arbitrary")),
    )(tile_e, h, wd)


@jax.jit
def kernel(hidden_states, Wr, Wg, Wu, Wd):
    b, s, d = hidden_states.shape
    x = hidden_states.reshape(s, d)
    (logits, rw, idx, w, dest, d1x, d2x, tile_e,
     wcat) = _router_dispatch(x, Wr)
    d1r = d1x.reshape(1, _SUB * _T)
    d2r = d2x.reshape(1, _SUB * _T)
    dT = dest.T
    tile_e1 = tile_e.reshape(_NT + 1)
    xs8, ws = _sc_scatter(x.reshape(_SUB * _T, _DC), d1r, d2r, wcat, dT)
    h = _e1(tile_e1, xs8.reshape(_NS, _D), Wg, Wu, ws)
    y = _e2(tile_e1, h, Wd)
    out8 = _sc_combine(y.reshape(_SUB * _NS, _DC), d1r, d2r)
    out = out8.reshape(_T, _D)
    return (out.reshape(b, s, d),
            logits.reshape(b, s, _E),
            idx.reshape(b, s, _K),
            w.reshape(b, s, _K),
            rw.reshape(b, s, _E))
